# baseline (speedup 1.0000x reference)
import jax
import jax.numpy as jnp
from jax import lax
from jax.experimental import pallas as pl
from jax.experimental.pallas import tpu as pltpu
from jax.experimental.pallas import tpu_sc as plsc

B, S, H = 128, 512, 768
NEG = float(jnp.finfo(jnp.float32).min)

B_SC = 64
B_TC = B - B_SC
R = 64
NG = H // 16
RB = 128
NB = S // RB


def _sc_body(t1_hbm, t2_hbm, lo_hbm, hi_hbm, inv_hbm, out_hbm,
             ids_v, inv_v, buf_v, out_v, sem):
    info = plsc.get_sparse_core_info()
    nc = info.num_cores
    ns = info.num_subcores
    nw = nc * ns
    wid = lax.axis_index("s") * nc + lax.axis_index("c")

    pltpu.sync_copy(lo_hbm, ids_v.at[0])
    pltpu.sync_copy(hi_hbm, ids_v.at[1])
    pltpu.sync_copy(inv_hbm, inv_v)

    def read_lane(ref, b):
        idx = jnp.zeros((16,), jnp.int32) + b
        return plsc.load_gather(ref, [idx])[0]

    units = []
    wid2 = lax.rem(wid + nw // 2, nw)
    for k in range(B_SC // nw):
        units.append((t1_hbm, wid + nw * k, True))
    for k in range(B_SC // nw):
        units.append((t2_hbm, wid2 + nw * k, False))

    meta = []
    for src, b, is_max in units:
        lo = read_lane(ids_v.at[0], b)
        hi = read_lane(ids_v.at[1], b)
        lo8 = lax.div(lo, 8) * 8
        nblk = lax.div(hi - lo8, R) + 1
        meta.append((src, b, is_max, lo, hi, lo8, nblk))

    def chunk_base(lo8, k):
        return jnp.minimum(lo8 + k * R, S - R)

    def start_chunk(src, b, base, par):
        pltpu.async_copy(src.at[b, pl.ds(base, R), :],
                         buf_v.at[par], sem.at[par])

    def wait_chunk(src, b, base, par):
        pltpu.make_async_copy(src.at[b, pl.ds(base, R), :],
                              buf_v.at[par], sem.at[par]).wait()

    start_chunk(meta[0][0], meta[0][1], chunk_base(meta[0][5], 0), 0)
    par0 = jnp.int32(0)

    for p, (src, b, is_max, lo, hi, lo8, nblk) in enumerate(meta):
        nxt = meta[p + 1] if p + 1 < len(meta) else None

        def chunk_body(k, accs, src=src, b=b, is_max=is_max, lo=lo, hi=hi,
                       lo8=lo8, nblk=nblk, nxt=nxt, par0=par0):
            par = lax.rem(par0 + k, 2)
            parn = lax.rem(par0 + k + 1, 2)

            @pl.when(k + 1 < nblk)
            def _():
                start_chunk(src, b, chunk_base(lo8, k + 1), parn)
            if nxt is not None:
                @pl.when(k + 1 == nblk)
                def _():
                    start_chunk(nxt[0], nxt[1], chunk_base(nxt[5], 0), parn)
            base = chunk_base(lo8, k)
            wait_chunk(src, b, base, par)

            GSUB = 16
            new_accs = list(accs)
            for g0 in range(0, NG, GSUB):
                if is_max:
                    def row(r, sub, g0=g0):
                        rc = jnp.clip(base + r, lo, hi) - base
                        return tuple(
                            jnp.maximum(sub[i],
                                        buf_v[par, rc, pl.ds(16 * (g0 + i), 16)])
                            for i in range(GSUB))
                else:
                    def row(r, sub, g0=g0):
                        pos = base + r
                        wlo = jnp.maximum(lo, lo8 + k * R)
                        w = jnp.where((pos >= wlo) & (pos <= hi),
                                      jnp.float32(1.0), jnp.float32(0.0))
                        return tuple(
                            sub[i] + buf_v[par, r, pl.ds(16 * (g0 + i), 16)] * w
                            for i in range(GSUB))
                sub = lax.fori_loop(0, R, row, tuple(accs[g0:g0 + GSUB]))
                new_accs[g0:g0 + GSUB] = list(sub)
            return tuple(new_accs)

        init = NEG if is_max else 0.0
        acc0 = tuple(jnp.full((16,), init, jnp.float32) for _ in range(NG))
        accs = lax.fori_loop(0, nblk, chunk_body, acc0)

        if is_max:
            for c in range(NG):
                out_v[pl.ds(16 * c, 16)] = accs[c]
            pltpu.sync_copy(out_v, out_hbm.at[b, pl.ds(0, H)])
        else:
            inv = read_lane(inv_v, b)
            for c in range(NG):
                out_v[pl.ds(16 * c, 16)] = accs[c] * inv
            pltpu.sync_copy(out_v, out_hbm.at[b, pl.ds(H, H)])
        par0 = lax.rem(par0 + nblk, 2)


def _sc_pool(tensor1, tensor2, lo, hi, inv):
    mesh = plsc.VectorSubcoreMesh(core_axis_name="c", subcore_axis_name="s")
    return pl.kernel(
        _sc_body,
        mesh=mesh,
        compiler_params=pltpu.CompilerParams(needs_layout_passes=False),
        out_type=jax.ShapeDtypeStruct((B_SC, 2 * H), jnp.float32),
        scratch_types=[
            pltpu.VMEM((2, B), jnp.int32),
            pltpu.VMEM((B,), jnp.float32),
            pltpu.VMEM((2, R, H), jnp.float32),
            pltpu.VMEM((H,), jnp.float32),
            pltpu.SemaphoreType.DMA((2,)),
        ],
    )(tensor1, tensor2, lo, hi, inv)


def _tc_body(lo_r, hi_r, inv_r, t1_ref, t2_ref, out_ref):
    b = pl.program_id(0) + B_SC
    j = pl.program_id(1)
    lo = lo_r[b]
    hi = hi_r[b]
    blk = jnp.minimum(lo // RB + j, hi // RB)
    in_span = j <= hi // RB - lo // RB

    @pl.when(in_span)
    def _():
        pos = blk * RB + lax.broadcasted_iota(jnp.int32, (RB, 1), 0)
        m = (pos >= lo) & (pos <= hi)
        part_max = jnp.max(jnp.where(m, t1_ref[0], NEG), axis=0)
        prev_max = jnp.where(j == 0, jnp.full((H,), NEG, jnp.float32),
                             out_ref[0, 0, :H])
        out_ref[0, 0, :H] = jnp.maximum(prev_max, part_max)
        part_sum = jnp.sum(jnp.where(m, t2_ref[0], 0.0), axis=0)
        prev_sum = jnp.where(j == 0, jnp.zeros((H,), jnp.float32),
                             out_ref[0, 0, H:])
        out_ref[0, 0, H:] = prev_sum + part_sum

    @pl.when(j == NB - 1)
    def _():
        out_ref[0, 0, H:] = out_ref[0, 0, H:] * inv_r[b]


def _tc_pool(tensor1, tensor2, lo, hi, inv):
    def tmap(b, j, lo_r, hi_r, inv_r):
        bb = b + B_SC
        return (bb, jnp.minimum(lo_r[bb] // RB + j, hi_r[bb] // RB), 0)

    def omap(b, j, lo_r, hi_r, inv_r):
        return (b, 0, 0)

    return pl.pallas_call(
        _tc_body,
        grid_spec=pltpu.PrefetchScalarGridSpec(
            num_scalar_prefetch=3,
            grid=(B_TC, NB),
            in_specs=[
                pl.BlockSpec((1, RB, H), tmap),
                pl.BlockSpec((1, RB, H), tmap),
            ],
            out_specs=pl.BlockSpec((1, 1, 2 * H), omap),
        ),
        out_shape=jax.ShapeDtypeStruct((B_TC, 1, 2 * H), jnp.float32),
    )(lo, hi, inv, tensor1, tensor2).reshape(B_TC, 2 * H)


@jax.jit
def _pooling(tensor1, tensor2, start_ids, end_ids):
    lo = jnp.minimum(start_ids, end_ids)
    hi = jnp.maximum(start_ids, end_ids)
    inv = 1.0 / (hi - lo + 1).astype(jnp.float32)
    sc = _sc_pool(tensor1, tensor2, lo, hi, inv)
    tc = _tc_pool(tensor1, tensor2, lo, hi, inv)
    return jnp.concatenate([sc, tc], axis=0)


def kernel(tensor1, tensor2, start_ids, end_ids):
    return _pooling(tensor1, tensor2, start_ids, end_ids)
